# baseline (speedup 1.0000x reference)
import functools

import jax
import jax.numpy as jnp
from jax import lax
from jax.experimental import pallas as pl
from jax.experimental.pallas import tpu as pltpu
from jax.experimental.pallas import tpu_sc as plsc

N_NODES = 10000
N_EDGES = 320000
IN_FEAT = 128
OUT_FEAT = 128
NUM_RELS = 8
NUM_BASES = 4

NC = 2
NS = 16
NW = NC * NS
E_PER_W = N_EDGES // NW
CHUNK = 80
N_CHUNKS = E_PER_W // CHUNK
ZROWS = 624
ZLAST = N_NODES - (NS - 1) * ZROWS
LANES = 16

ROW_BLK = 2000
N_ROW_BLKS = N_NODES // ROW_BLK


def _z_body(coef_ref, x_ref, basis_ref, z_ref):
    r = pl.program_id(0)
    rr = pl.multiple_of(r * (IN_FEAT // NUM_RELS), 8)
    w = jnp.zeros((IN_FEAT, OUT_FEAT), jnp.float32)
    for b in range(NUM_BASES):
        sb = basis_ref[b, pl.ds(rr, IN_FEAT // NUM_RELS), :]
        eb = jnp.broadcast_to(
            sb[:, None, :], (IN_FEAT // NUM_RELS, NUM_RELS, OUT_FEAT)
        ).reshape(IN_FEAT, OUT_FEAT)
        w = w + coef_ref[:, b:b + 1] * eb
    z_ref[...] = jnp.dot(x_ref[...], w, preferred_element_type=jnp.float32)


_z_call = pl.pallas_call(
    _z_body,
    grid=(NUM_RELS, N_ROW_BLKS),
    in_specs=[
        pl.BlockSpec((IN_FEAT, NUM_BASES), lambda r, i: (0, 0)),
        pl.BlockSpec((ROW_BLK, IN_FEAT), lambda r, i: (i, 0)),
        pl.BlockSpec((NUM_BASES, IN_FEAT, OUT_FEAT), lambda r, i: (0, 0, 0)),
    ],
    out_specs=pl.BlockSpec((ROW_BLK, OUT_FEAT),
                           lambda r, i: (r * N_ROW_BLKS + i, 0)),
    out_shape=jax.ShapeDtypeStruct((NUM_RELS * N_NODES, OUT_FEAT), jnp.float32),
)


_sc_mesh = plsc.VectorSubcoreMesh(core_axis_name="c", subcore_axis_name="s")


NBUF = 3


@functools.partial(
    pl.kernel,
    out_type=jax.ShapeDtypeStruct((NC, N_NODES, OUT_FEAT), jnp.float32),
    mesh=_sc_mesh,
    compiler_params=pltpu.CompilerParams(needs_layout_passes=False),
    scratch_types=(
        [pltpu.VMEM((CHUNK, OUT_FEAT), jnp.float32)] * NBUF
        + [pltpu.VMEM((CHUNK,), jnp.int32)] * NBUF
        + [pltpu.VMEM((CHUNK,), jnp.int32)] * NBUF
        + [pltpu.VMEM((CHUNK,), jnp.int32)] * NBUF
        + [pltpu.VMEM((CHUNK,), jnp.float32)] * NBUF
        + [pltpu.VMEM((CHUNK,), jnp.int32)] * NBUF
        + [pltpu.VMEM_SHARED((N_NODES, OUT_FEAT), jnp.float32)]
        + [pltpu.SemaphoreType.DMA] * (3 * NBUF)
    ),
)
def _sc_scatter(z_hbm, src_hbm, dst_hbm, et_hbm, ew_hbm, zeros_hbm, p_hbm,
                r0, r1, r2, z0, z1, z2, s0, s1, s2, t0, t1, t2,
                w0, w1, w2, d0, d1, d2, accum,
                k0, k1, k2, g0, g1, g2, x0, x1, x2):
    rows = (r0, r1, r2)
    zix = (z0, z1, z2)
    sv = (s0, s1, s2)
    tv = (t0, t1, t2)
    wv = (w0, w1, w2)
    dv = (d0, d1, d2)
    ksem = (k0, k1, k2)
    gsem = (g0, g1, g2)
    ssem = (x0, x1, x2)

    c = lax.axis_index("c")
    s = lax.axis_index("s")
    wid = s * NC + c
    ebase = wid * E_PER_W
    rbase = pl.multiple_of(s * ZROWS, 8)

    @pl.when(s < NS - 1)
    def _zero_main():
        pltpu.sync_copy(zeros_hbm.at[pl.ds(0, ZROWS)],
                        accum.at[pl.ds(rbase, ZROWS)])

    @pl.when(s == NS - 1)
    def _zero_last():
        pltpu.sync_copy(zeros_hbm, accum.at[pl.ds(rbase, ZLAST)])

    def small_start(ci, b):
        off = ebase + ci * CHUNK
        pltpu.async_copy(src_hbm.at[pl.ds(off, CHUNK)], sv[b], ksem[b])
        pltpu.async_copy(et_hbm.at[pl.ds(off, CHUNK)], tv[b], ksem[b])
        pltpu.async_copy(ew_hbm.at[pl.ds(off, CHUNK)], wv[b], ksem[b])
        pltpu.async_copy(dst_hbm.at[pl.ds(off, CHUNK)], dv[b], ksem[b])

    def small_wait(ci, b):
        off = ebase + ci * CHUNK
        pltpu.make_async_copy(src_hbm.at[pl.ds(off, CHUNK)], sv[b],
                              ksem[b]).wait()
        pltpu.make_async_copy(et_hbm.at[pl.ds(off, CHUNK)], tv[b],
                              ksem[b]).wait()
        pltpu.make_async_copy(ew_hbm.at[pl.ds(off, CHUNK)], wv[b],
                              ksem[b]).wait()
        pltpu.make_async_copy(dst_hbm.at[pl.ds(off, CHUNK)], dv[b],
                              ksem[b]).wait()

    def zix_compute(b):
        for k in range(CHUNK // LANES):
            sl = pl.ds(k * LANES, LANES)
            zix[b][sl] = tv[b][sl] * N_NODES + sv[b][sl]

    def gather_start(b):
        pltpu.async_copy(z_hbm.at[zix[b]], rows[b], gsem[b])

    def gather_wait(b):
        pltpu.make_async_copy(z_hbm.at[zix[b]], rows[b], gsem[b]).wait()

    def scale(b):
        def scale_body(k, carry2):
            w16 = wv[b][pl.ds(k * LANES, LANES)]
            for j in range(LANES):
                wb = w16[jnp.full((LANES,), j, jnp.int32)]
                e = k * LANES + j
                for f in range(OUT_FEAT // LANES):
                    sl = pl.ds(f * LANES, LANES)
                    rows[b][e, sl] = rows[b][e, sl] * wb
            return carry2

        lax.fori_loop(0, CHUNK // LANES, scale_body, 0)

    def scatter_start(b):
        pltpu.async_copy(rows[b], accum.at[dv[b]], ssem[b], add=True)

    def scatter_wait(b):
        pltpu.make_async_copy(rows[b], accum.at[dv[b]], ssem[b]).wait()

    small_start(0, 0)
    small_start(1, 1)
    small_wait(0, 0)
    zix_compute(0)
    plsc.subcore_barrier()
    gather_start(0)

    def chunk_step(ci, b):
        b1 = (b + 1) % NBUF
        b2 = (b + 2) % NBUF

        @pl.when(ci + 1 < N_CHUNKS)
        def _():
            small_wait(ci + 1, b1)
            zix_compute(b1)

        @pl.when(ci >= 1)
        def _():
            scatter_wait(b2)

        @pl.when(ci + 2 < N_CHUNKS)
        def _():
            small_start(ci + 2, b2)

        gather_wait(b)
        scale(b)

        @pl.when(ci + 1 < N_CHUNKS)
        def _():
            gather_start(b1)

        scatter_start(b)

    def tri_body(i, carry):
        for b in range(NBUF):
            ci = NBUF * i + b

            @pl.when(ci < N_CHUNKS)
            def _():
                chunk_step(ci, b)

        return carry

    lax.fori_loop(0, (N_CHUNKS + NBUF - 1) // NBUF, tri_body, 0)
    scatter_wait((N_CHUNKS - 1) % NBUF)
    plsc.subcore_barrier()

    @pl.when(s < NS - 1)
    def _out_main():
        pltpu.sync_copy(accum.at[pl.ds(rbase, ZROWS)],
                        p_hbm.at[c, pl.ds(rbase, ZROWS)])

    @pl.when(s == NS - 1)
    def _out_last():
        pltpu.sync_copy(accum.at[pl.ds(rbase, ZLAST)],
                        p_hbm.at[c, pl.ds(rbase, ZLAST)])


def _add_body(p_ref, o_ref):
    o_ref[...] = p_ref[0] + p_ref[1]


_add_call = pl.pallas_call(
    _add_body,
    grid=(N_ROW_BLKS,),
    in_specs=[pl.BlockSpec((NC, ROW_BLK, OUT_FEAT), lambda i: (0, i, 0))],
    out_specs=pl.BlockSpec((ROW_BLK, OUT_FEAT), lambda i: (i, 0)),
    out_shape=jax.ShapeDtypeStruct((N_NODES, OUT_FEAT), jnp.float32),
)


def kernel(x, edge_index, edge_type, edge_weight, basis_weights, w_comp):
    coef = jnp.tile(w_comp, (IN_FEAT // NUM_RELS, 1))
    z = _z_call(coef, x, basis_weights)
    zeros = jnp.zeros((ZLAST, OUT_FEAT), jnp.float32)
    p = _sc_scatter(z, edge_index[0], edge_index[1], edge_type, edge_weight,
                    zeros)
    return _add_call(p)


# speedup vs baseline: 1.3842x; 1.3842x over previous
import functools

import jax
import jax.numpy as jnp
from jax import lax
from jax.experimental import pallas as pl
from jax.experimental.pallas import tpu as pltpu
from jax.experimental.pallas import tpu_sc as plsc

N_NODES = 10000
N_EDGES = 320000
IN_FEAT = 128
OUT_FEAT = 128
NUM_RELS = 8
NUM_BASES = 4

NC = 2
NS = 16
NW = NC * NS
E_PER_W = N_EDGES // NW
CHUNK = 80
N_CHUNKS = E_PER_W // CHUNK
ZROWS = 624
ZLAST = N_NODES - (NS - 1) * ZROWS
LANES = 16

ROW_BLK = 2000
N_ROW_BLKS = N_NODES // ROW_BLK


def _z_body(coef_ref, x_ref, basis_ref, z_ref):
    r = pl.program_id(0)
    rr = pl.multiple_of(r * (IN_FEAT // NUM_RELS), 8)
    w = jnp.zeros((IN_FEAT, OUT_FEAT), jnp.float32)
    for b in range(NUM_BASES):
        sb = basis_ref[b, pl.ds(rr, IN_FEAT // NUM_RELS), :]
        eb = jnp.broadcast_to(
            sb[:, None, :], (IN_FEAT // NUM_RELS, NUM_RELS, OUT_FEAT)
        ).reshape(IN_FEAT, OUT_FEAT)
        w = w + coef_ref[:, b:b + 1] * eb
    z_ref[...] = jnp.dot(x_ref[...], w, preferred_element_type=jnp.float32)


_z_call = pl.pallas_call(
    _z_body,
    grid=(NUM_RELS, N_ROW_BLKS),
    in_specs=[
        pl.BlockSpec((IN_FEAT, NUM_BASES), lambda r, i: (0, 0)),
        pl.BlockSpec((ROW_BLK, IN_FEAT), lambda r, i: (i, 0)),
        pl.BlockSpec((NUM_BASES, IN_FEAT, OUT_FEAT), lambda r, i: (0, 0, 0)),
    ],
    out_specs=pl.BlockSpec((ROW_BLK, OUT_FEAT),
                           lambda r, i: (r * N_ROW_BLKS + i, 0)),
    out_shape=jax.ShapeDtypeStruct((NUM_RELS * N_NODES, OUT_FEAT), jnp.float32),
)


_sc_mesh = plsc.VectorSubcoreMesh(core_axis_name="c", subcore_axis_name="s")


NBUF = 4


@functools.partial(
    pl.kernel,
    out_type=jax.ShapeDtypeStruct((NC, N_NODES, OUT_FEAT), jnp.float32),
    mesh=_sc_mesh,
    compiler_params=pltpu.CompilerParams(needs_layout_passes=False),
    scratch_types=(
        [pltpu.VMEM((CHUNK, OUT_FEAT), jnp.float32)] * NBUF
        + [pltpu.VMEM((CHUNK,), jnp.int32)] * NBUF
        + [pltpu.VMEM((CHUNK,), jnp.int32)] * NBUF
        + [pltpu.VMEM((CHUNK,), jnp.int32)] * NBUF
        + [pltpu.VMEM((CHUNK,), jnp.float32)] * NBUF
        + [pltpu.VMEM((CHUNK,), jnp.int32)] * NBUF
        + [pltpu.VMEM_SHARED((N_NODES, OUT_FEAT), jnp.float32)]
        + [pltpu.SemaphoreType.DMA] * (4 * NBUF)
    ),
)
def _sc_scatter(z_hbm, src_hbm, dst_hbm, et_hbm, ew_hbm, zeros_hbm, p_hbm,
                r0, r1, r2, r3, z0, z1, z2, z3, s0, s1, s2, s3,
                t0, t1, t2, t3, w0, w1, w2, w3, d0, d1, d2, d3, accum,
                k0, k1, k2, k3, e0, e1, e2, e3,
                g0, g1, g2, g3, x0, x1, x2, x3):
    rows = (r0, r1, r2, r3)
    zix = (z0, z1, z2, z3)
    sv = (s0, s1, s2, s3)
    tv = (t0, t1, t2, t3)
    wv = (w0, w1, w2, w3)
    dv = (d0, d1, d2, d3)
    ksem = (k0, k1, k2, k3)
    dsem = (e0, e1, e2, e3)
    gsem = (g0, g1, g2, g3)
    ssem = (x0, x1, x2, x3)

    c = lax.axis_index("c")
    s = lax.axis_index("s")
    wid = s * NC + c
    ebase = wid * E_PER_W
    rbase = pl.multiple_of(s * ZROWS, 8)

    @pl.when(s < NS - 1)
    def _zero_main():
        pltpu.sync_copy(zeros_hbm.at[pl.ds(0, ZROWS)],
                        accum.at[pl.ds(rbase, ZROWS)])

    @pl.when(s == NS - 1)
    def _zero_last():
        pltpu.sync_copy(zeros_hbm, accum.at[pl.ds(rbase, ZLAST)])

    def stw_start(ci, b):
        off = ebase + ci * CHUNK
        pltpu.async_copy(src_hbm.at[pl.ds(off, CHUNK)], sv[b], ksem[b])
        pltpu.async_copy(et_hbm.at[pl.ds(off, CHUNK)], tv[b], ksem[b])
        pltpu.async_copy(ew_hbm.at[pl.ds(off, CHUNK)], wv[b], ksem[b])

    def stw_wait(ci, b):
        off = ebase + ci * CHUNK
        pltpu.make_async_copy(src_hbm.at[pl.ds(off, CHUNK)], sv[b],
                              ksem[b]).wait()
        pltpu.make_async_copy(et_hbm.at[pl.ds(off, CHUNK)], tv[b],
                              ksem[b]).wait()
        pltpu.make_async_copy(ew_hbm.at[pl.ds(off, CHUNK)], wv[b],
                              ksem[b]).wait()

    def dv_start(ci, b):
        off = ebase + ci * CHUNK
        pltpu.async_copy(dst_hbm.at[pl.ds(off, CHUNK)], dv[b], dsem[b])

    def dv_wait(ci, b):
        off = ebase + ci * CHUNK
        pltpu.make_async_copy(dst_hbm.at[pl.ds(off, CHUNK)], dv[b],
                              dsem[b]).wait()

    def zix_compute(b):
        for k in range(CHUNK // LANES):
            sl = pl.ds(k * LANES, LANES)
            zix[b][sl] = tv[b][sl] * N_NODES + sv[b][sl]

    def gather_start(b):
        pltpu.async_copy(z_hbm.at[zix[b]], rows[b], gsem[b])

    def gather_wait(b):
        pltpu.make_async_copy(z_hbm.at[zix[b]], rows[b], gsem[b]).wait()

    def scale(b):
        def scale_body(k, carry2):
            w16 = wv[b][pl.ds(k * LANES, LANES)]
            for j in range(LANES):
                wb = w16[jnp.full((LANES,), j, jnp.int32)]
                e = k * LANES + j
                for f in range(OUT_FEAT // LANES):
                    sl = pl.ds(f * LANES, LANES)
                    rows[b][e, sl] = rows[b][e, sl] * wb
            return carry2

        lax.fori_loop(0, CHUNK // LANES, scale_body, 0)

    def scatter_start(b):
        pltpu.async_copy(rows[b], accum.at[dv[b]], ssem[b], add=True)

    def scatter_wait(b):
        pltpu.make_async_copy(rows[b], accum.at[dv[b]], ssem[b]).wait()

    stw_start(0, 0)
    stw_start(1, 1)
    stw_start(2, 2)
    dv_start(0, 0)
    dv_start(1, 1)
    stw_wait(0, 0)
    zix_compute(0)
    stw_wait(1, 1)
    zix_compute(1)
    plsc.subcore_barrier()
    gather_start(0)

    def chunk_step(ci, b):
        b1 = (b + 1) % NBUF
        b2 = (b + 2) % NBUF
        b3 = (b + 3) % NBUF

        @pl.when(ci >= 2)
        def _():
            scatter_wait(b2)

        @pl.when(ci + 1 < N_CHUNKS)
        def _():
            gather_start(b1)

        @pl.when(ci + 3 < N_CHUNKS)
        def _():
            stw_start(ci + 3, b3)

        @pl.when(ci + 2 < N_CHUNKS)
        def _():
            dv_start(ci + 2, b2)
            stw_wait(ci + 2, b2)
            zix_compute(b2)

        gather_wait(b)
        dv_wait(ci, b)
        scale(b)
        scatter_start(b)

    def quad_body(i, carry):
        for b in range(NBUF):
            ci = NBUF * i + b

            @pl.when(ci < N_CHUNKS)
            def _():
                chunk_step(ci, b)

        return carry

    lax.fori_loop(0, (N_CHUNKS + NBUF - 1) // NBUF, quad_body, 0)
    scatter_wait((N_CHUNKS - 2) % NBUF)
    scatter_wait((N_CHUNKS - 1) % NBUF)
    plsc.subcore_barrier()

    @pl.when(s < NS - 1)
    def _out_main():
        pltpu.sync_copy(accum.at[pl.ds(rbase, ZROWS)],
                        p_hbm.at[c, pl.ds(rbase, ZROWS)])

    @pl.when(s == NS - 1)
    def _out_last():
        pltpu.sync_copy(accum.at[pl.ds(rbase, ZLAST)],
                        p_hbm.at[c, pl.ds(rbase, ZLAST)])


def _add_body(p_ref, o_ref):
    o_ref[...] = p_ref[0] + p_ref[1]


_add_call = pl.pallas_call(
    _add_body,
    grid=(N_ROW_BLKS,),
    in_specs=[pl.BlockSpec((NC, ROW_BLK, OUT_FEAT), lambda i: (0, i, 0))],
    out_specs=pl.BlockSpec((ROW_BLK, OUT_FEAT), lambda i: (i, 0)),
    out_shape=jax.ShapeDtypeStruct((N_NODES, OUT_FEAT), jnp.float32),
)


def kernel(x, edge_index, edge_type, edge_weight, basis_weights, w_comp):
    coef = jnp.tile(w_comp, (IN_FEAT // NUM_RELS, 1))
    z = _z_call(coef, x, basis_weights)
    zeros = jnp.zeros((ZLAST, OUT_FEAT), jnp.float32)
    p = _sc_scatter(z, edge_index[0], edge_index[1], edge_type, edge_weight,
                    zeros)
    return _add_call(p)


# speedup vs baseline: 1.5081x; 1.0895x over previous
import functools

import jax
import jax.numpy as jnp
from jax import lax
from jax.experimental import pallas as pl
from jax.experimental.pallas import tpu as pltpu
from jax.experimental.pallas import tpu_sc as plsc

N_NODES = 10000
N_EDGES = 320000
IN_FEAT = 128
OUT_FEAT = 128
NUM_RELS = 8
NUM_BASES = 4

NC = 2
NS = 16
NW = NC * NS
E_PER_W = N_EDGES // NW
CHUNK = 80
N_CHUNKS = E_PER_W // CHUNK
ZROWS = 624
ZLAST = N_NODES - (NS - 1) * ZROWS
LANES = 16

ROW_BLK = 2000
N_ROW_BLKS = N_NODES // ROW_BLK


def _z_body(coef_ref, x_ref, basis_ref, z_ref):
    r = pl.program_id(1)
    rr = pl.multiple_of(r * (IN_FEAT // NUM_RELS), 8)
    w = jnp.zeros((IN_FEAT, OUT_FEAT), jnp.float32)
    for b in range(NUM_BASES):
        sb = basis_ref[b, pl.ds(rr, IN_FEAT // NUM_RELS), :]
        eb = jnp.broadcast_to(
            sb[:, None, :], (IN_FEAT // NUM_RELS, NUM_RELS, OUT_FEAT)
        ).reshape(IN_FEAT, OUT_FEAT)
        w = w + coef_ref[:, b:b + 1] * eb
    z_ref[...] = jnp.dot(x_ref[...], w, preferred_element_type=jnp.float32)


_z_call = pl.pallas_call(
    _z_body,
    grid=(N_ROW_BLKS, NUM_RELS),
    in_specs=[
        pl.BlockSpec((IN_FEAT, NUM_BASES), lambda i, r: (0, 0)),
        pl.BlockSpec((ROW_BLK, IN_FEAT), lambda i, r: (i, 0)),
        pl.BlockSpec((NUM_BASES, IN_FEAT, OUT_FEAT), lambda i, r: (0, 0, 0)),
    ],
    out_specs=pl.BlockSpec((ROW_BLK, OUT_FEAT),
                           lambda i, r: (r * N_ROW_BLKS + i, 0)),
    out_shape=jax.ShapeDtypeStruct((NUM_RELS * N_NODES, OUT_FEAT), jnp.float32),
)


_sc_mesh = plsc.VectorSubcoreMesh(core_axis_name="c", subcore_axis_name="s")

NBUF = 4


@functools.partial(
    pl.kernel,
    out_type=jax.ShapeDtypeStruct((NC, N_NODES, OUT_FEAT), jnp.float32),
    mesh=_sc_mesh,
    compiler_params=pltpu.CompilerParams(needs_layout_passes=False),
    scratch_types=(
        [pltpu.VMEM((CHUNK, OUT_FEAT), jnp.float32)] * NBUF
        + [pltpu.VMEM((CHUNK,), jnp.int32)] * NBUF
        + [pltpu.VMEM((CHUNK,), jnp.float32)] * NBUF
        + [pltpu.VMEM((CHUNK,), jnp.int32)] * NBUF
        + [pltpu.VMEM_SHARED((N_NODES, OUT_FEAT), jnp.float32)]
        + [pltpu.SemaphoreType.DMA] * (4 * NBUF)
    ),
)
def _sc_scatter(z_hbm, zidx_hbm, dst_hbm, ew_hbm, zeros_hbm, p_hbm,
                r0, r1, r2, r3, z0, z1, z2, z3,
                w0, w1, w2, w3, d0, d1, d2, d3, accum,
                k0, k1, k2, k3, e0, e1, e2, e3,
                g0, g1, g2, g3, x0, x1, x2, x3):
    rows = (r0, r1, r2, r3)
    zix = (z0, z1, z2, z3)
    wv = (w0, w1, w2, w3)
    dv = (d0, d1, d2, d3)
    ksem = (k0, k1, k2, k3)
    dsem = (e0, e1, e2, e3)
    gsem = (g0, g1, g2, g3)
    ssem = (x0, x1, x2, x3)

    c = lax.axis_index("c")
    s = lax.axis_index("s")
    wid = s * NC + c
    ebase = wid * E_PER_W
    rbase = pl.multiple_of(s * ZROWS, 8)

    @pl.when(s < NS - 1)
    def _zero_main():
        pltpu.sync_copy(zeros_hbm.at[pl.ds(0, ZROWS)],
                        accum.at[pl.ds(rbase, ZROWS)])

    @pl.when(s == NS - 1)
    def _zero_last():
        pltpu.sync_copy(zeros_hbm, accum.at[pl.ds(rbase, ZLAST)])

    def zw_start(ci, b):
        off = ebase + ci * CHUNK
        pltpu.async_copy(zidx_hbm.at[pl.ds(off, CHUNK)], zix[b], ksem[b])
        pltpu.async_copy(ew_hbm.at[pl.ds(off, CHUNK)], wv[b], ksem[b])

    def zw_wait(ci, b):
        off = ebase + ci * CHUNK
        pltpu.make_async_copy(zidx_hbm.at[pl.ds(off, CHUNK)], zix[b],
                              ksem[b]).wait()
        pltpu.make_async_copy(ew_hbm.at[pl.ds(off, CHUNK)], wv[b],
                              ksem[b]).wait()

    def dv_start(ci, b):
        off = ebase + ci * CHUNK
        pltpu.async_copy(dst_hbm.at[pl.ds(off, CHUNK)], dv[b], dsem[b])

    def dv_wait(ci, b):
        off = ebase + ci * CHUNK
        pltpu.make_async_copy(dst_hbm.at[pl.ds(off, CHUNK)], dv[b],
                              dsem[b]).wait()

    def gather_start(b):
        pltpu.async_copy(z_hbm.at[zix[b]], rows[b], gsem[b])

    def gather_wait(b):
        pltpu.make_async_copy(z_hbm.at[zix[b]], rows[b], gsem[b]).wait()

    def scale(b):
        def scale_body(k, carry2):
            w16 = wv[b][pl.ds(k * LANES, LANES)]
            for j in range(LANES):
                wb = w16[jnp.full((LANES,), j, jnp.int32)]
                e = k * LANES + j
                for f in range(OUT_FEAT // LANES):
                    sl = pl.ds(f * LANES, LANES)
                    rows[b][e, sl] = rows[b][e, sl] * wb
            return carry2

        lax.fori_loop(0, CHUNK // LANES, scale_body, 0)

    def scatter_start(b):
        pltpu.async_copy(rows[b], accum.at[dv[b]], ssem[b], add=True)

    def scatter_wait(b):
        pltpu.make_async_copy(rows[b], accum.at[dv[b]], ssem[b]).wait()

    zw_start(0, 0)
    zw_start(1, 1)
    zw_start(2, 2)
    dv_start(0, 0)
    dv_start(1, 1)
    zw_wait(0, 0)
    plsc.subcore_barrier()
    gather_start(0)
    zw_wait(1, 1)
    gather_start(1)

    def chunk_step(ci, b):
        b2 = (b + 2) % NBUF
        b3 = (b + 3) % NBUF

        @pl.when(ci >= 2)
        def _():
            scatter_wait(b2)

        @pl.when(ci + 2 < N_CHUNKS)
        def _():
            dv_start(ci + 2, b2)
            zw_wait(ci + 2, b2)
            gather_start(b2)

        @pl.when(ci + 3 < N_CHUNKS)
        def _():
            zw_start(ci + 3, b3)

        gather_wait(b)
        dv_wait(ci, b)
        scale(b)
        scatter_start(b)

    def quad_body(i, carry):
        for b in range(NBUF):
            ci = NBUF * i + b

            @pl.when(ci < N_CHUNKS)
            def _():
                chunk_step(ci, b)

        return carry

    lax.fori_loop(0, (N_CHUNKS + NBUF - 1) // NBUF, quad_body, 0)
    scatter_wait((N_CHUNKS - 2) % NBUF)
    scatter_wait((N_CHUNKS - 1) % NBUF)
    plsc.subcore_barrier()

    @pl.when(s < NS - 1)
    def _out_main():
        pltpu.sync_copy(accum.at[pl.ds(rbase, ZROWS)],
                        p_hbm.at[c, pl.ds(rbase, ZROWS)])

    @pl.when(s == NS - 1)
    def _out_last():
        pltpu.sync_copy(accum.at[pl.ds(rbase, ZLAST)],
                        p_hbm.at[c, pl.ds(rbase, ZLAST)])


def _add_body(p_ref, o_ref):
    o_ref[...] = p_ref[0] + p_ref[1]


_add_call = pl.pallas_call(
    _add_body,
    grid=(N_ROW_BLKS,),
    in_specs=[pl.BlockSpec((NC, ROW_BLK, OUT_FEAT), lambda i: (0, i, 0))],
    out_specs=pl.BlockSpec((ROW_BLK, OUT_FEAT), lambda i: (i, 0)),
    out_shape=jax.ShapeDtypeStruct((N_NODES, OUT_FEAT), jnp.float32),
)


def kernel(x, edge_index, edge_type, edge_weight, basis_weights, w_comp):
    coef = jnp.tile(w_comp, (IN_FEAT // NUM_RELS, 1))
    z = _z_call(coef, x, basis_weights)
    zidx = edge_type * jnp.int32(N_NODES) + edge_index[0]
    zeros = jnp.zeros((ZLAST, OUT_FEAT), jnp.float32)
    p = _sc_scatter(z, zidx, edge_index[1], edge_weight, zeros)
    return _add_call(p)
